# baseline (speedup 1.0000x reference)
import functools

import jax
import jax.numpy as jnp
from jax import lax
from jax.experimental import pallas as pl
from jax.experimental.pallas import tpu as pltpu, tpu_sc as plsc

FRAME_NUM = 100000
T_DIM = 64
BATCH = 16384

_info = plsc.get_sparse_core_info()
_NC, _NS = _info.num_cores, _info.num_subcores
_NW = _NC * _NS
_B_PER_W = BATCH // _NW


@functools.partial(
    pl.kernel,
    mesh=plsc.VectorSubcoreMesh(core_axis_name="c", subcore_axis_name="s"),
    out_type=jax.ShapeDtypeStruct((BATCH, T_DIM), jnp.float32),
    scratch_types=[
        pltpu.VMEM((_B_PER_W,), jnp.int32),
        pltpu.VMEM((_B_PER_W, T_DIM), jnp.float32),
        pltpu.SemaphoreType.DMA,
        pltpu.SemaphoreType.DMA,
    ],
    compiler_params=pltpu.CompilerParams(use_tc_tiling_on_sc=True),
)
def _gather_kernel(table_hbm, idx_hbm, out_hbm, idx_v, rows_v, sem_i, sem):
    wid = lax.axis_index("s") * _NC + lax.axis_index("c")
    base = wid * _B_PER_W
    pltpu.async_copy(idx_hbm.at[pl.ds(base, _B_PER_W)], idx_v, sem_i).wait()

    def fire(g, _):
        v = idx_v[pl.ds(g * 16, 16)]
        for i in range(16):
            r = v[i]
            pltpu.async_copy(table_hbm.at[r], rows_v.at[g * 16 + i], sem)
        return _

    lax.fori_loop(0, _B_PER_W // 16, fire, None)

    def drain(j, _):
        pltpu.make_async_copy(table_hbm.at[0], rows_v.at[j], sem).wait()
        return _

    lax.fori_loop(0, _B_PER_W, drain, None)
    pltpu.sync_copy(rows_v, out_hbm.at[pl.ds(base, _B_PER_W)])


def kernel(t_idx, t_codes):
    return _gather_kernel(t_codes, t_idx)
